# Initial kernel scaffold; baseline (speedup 1.0000x reference)
import jax
import jax.numpy as jnp
from jax.experimental import pallas as pl


def kernel(x, edge_index, W1, b1, W2, b2, W3, b3):
    raise NotImplementedError("write your pallas kernel here")


# speedup vs baseline: 5.3472x; 5.3472x over previous
import functools

import jax
import jax.numpy as jnp
from jax import lax
from jax.experimental import pallas as pl
from jax.experimental.pallas import tpu as pltpu
from jax.experimental.pallas import tpu_sc as plsc

N = 100000
E = 1600000

NC = 2
NS = 16
NW = NC * NS
EPW = E // NW
EB = 80
NCHUNK = EPW // EB
NP = 100096
RPT = NP // NS

_mesh = plsc.VectorSubcoreMesh(core_axis_name="c", subcore_axis_name="s")


def _f32(shape):
    return jax.ShapeDtypeStruct(shape, jnp.float32)


def _make_deg_kernel():
    @functools.partial(
        pl.kernel,
        out_type=_f32((NC * NP, 4)),
        mesh=_mesh,
        compiler_params=pltpu.CompilerParams(use_tc_tiling_on_sc=False),
        scratch_types=[
            pltpu.VMEM((EB,), jnp.int32),
            pltpu.VMEM((EB, 4), jnp.float32),
            pltpu.VMEM((RPT, 4), jnp.float32),
            pltpu.VMEM_SHARED((NP, 4), jnp.float32),
        ],
    )
    def deg_kernel(dst_hbm, ones_hbm, zeros_hbm, out_hbm, didx, ones_v, wbuf, zsh):
        c = lax.axis_index("c")
        s = lax.axis_index("s")
        wid = s * NC + c
        pltpu.sync_copy(zeros_hbm, wbuf)
        pltpu.sync_copy(wbuf, zsh.at[pl.ds(s * RPT, RPT)])
        pltpu.sync_copy(ones_hbm, ones_v)
        plsc.subcore_barrier()
        base0 = wid * EPW

        def step(i, carry):
            base = base0 + i * EB
            pltpu.sync_copy(dst_hbm.at[pl.ds(base, EB)], didx)
            pltpu.sync_copy(ones_v, zsh.at[didx], add=True)
            return carry

        lax.fori_loop(0, NCHUNK, step, 0)
        plsc.subcore_barrier()
        pltpu.sync_copy(zsh.at[pl.ds(s * RPT, RPT)], wbuf)
        pltpu.sync_copy(wbuf, out_hbm.at[pl.ds(c * NP + s * RPT, RPT)])

    return deg_kernel


def _make_prop_kernel(fc, n_tables):
    @functools.partial(
        pl.kernel,
        out_type=[_f32((NC * NP, fc)) for _ in range(n_tables)],
        mesh=_mesh,
        compiler_params=pltpu.CompilerParams(use_tc_tiling_on_sc=False),
        scratch_types=[
            pltpu.VMEM((EB,), jnp.int32),
            pltpu.VMEM((EB,), jnp.int32),
            pltpu.VMEM((EB, fc), jnp.float32),
            pltpu.VMEM((RPT, fc), jnp.float32),
            pltpu.VMEM_SHARED((NP, fc), jnp.float32),
            pltpu.SemaphoreType.DMA,
        ],
    )
    def prop_kernel(src_hbm, dst_hbm, *rest):
        tabs = rest[:n_tables]
        zeros_hbm = rest[n_tables]
        outs = rest[n_tables + 1:2 * n_tables + 1]
        sidx, didx, rows, wbuf, zsh, gsem = rest[2 * n_tables + 1:]
        c = lax.axis_index("c")
        s = lax.axis_index("s")
        wid = s * NC + c
        base0 = wid * EPW
        for t in range(n_tables):
            pltpu.sync_copy(zeros_hbm, wbuf)
            pltpu.sync_copy(wbuf, zsh.at[pl.ds(s * RPT, RPT)])
            plsc.subcore_barrier()

            def step(i, carry):
                base = base0 + i * EB
                pltpu.sync_copy(src_hbm.at[pl.ds(base, EB)], sidx)
                pltpu.sync_copy(dst_hbm.at[pl.ds(base, EB)], didx)
                pltpu.async_copy(tabs[t].at[sidx], rows, gsem).wait()
                pltpu.sync_copy(rows, zsh.at[didx], add=True)
                return carry

            lax.fori_loop(0, NCHUNK, step, 0)
            plsc.subcore_barrier()
            pltpu.sync_copy(zsh.at[pl.ds(s * RPT, RPT)], wbuf)
            pltpu.sync_copy(wbuf, outs[t].at[pl.ds(c * NP + s * RPT, RPT)])
            if t + 1 < n_tables:
                plsc.subcore_barrier()

    return prop_kernel


_deg_kernel = _make_deg_kernel()
_prop4 = _make_prop_kernel(4, 1)
_prop8x4 = _make_prop_kernel(8, 4)
_prop8x2 = _make_prop_kernel(8, 2)


_BLK = 2000
_GRID = N // _BLK


def _row_spec(f):
    return pl.BlockSpec((_BLK, f), lambda i: (i, 0))


def _full_spec(a, b):
    return pl.BlockSpec((a, b), lambda i: (0, 0))


def _stage_a_body(d0, d1, x, dinv, y1):
    deg = d0[:, 0:1] + d1[:, 0:1] + 1.0
    dv = lax.rsqrt(deg)
    dinv[...] = dv
    y1[...] = x[...] * dv


def _stage_b_body(z0, z1, y1, dinv, w1, b1, y2a, y2b, y2c, y2d):
    zz = z0[...] + z1[...] + y1[...]
    p1 = zz * dinv[...]
    h1 = jnp.maximum(jnp.dot(p1, w1[...], preferred_element_type=jnp.float32)
                     + b1[...], 0.0)
    y2 = h1 * dinv[...]
    y2a[...] = y2[:, 0:8]
    y2b[...] = y2[:, 8:16]
    y2c[...] = y2[:, 16:24]
    y2d[...] = y2[:, 24:32]


def _stage_c_body(za0, za1, zb0, zb1, zc0, zc1, zd0, zd1,
                  y2a, y2b, y2c, y2d, dinv, w2a, w2b, w2c, w2d, w3, b2,
                  y3a, y3b):
    dv = dinv[...]
    p2a = (za0[...] + za1[...] + y2a[...]) * dv
    p2b = (zb0[...] + zb1[...] + y2b[...]) * dv
    p2c = (zc0[...] + zc1[...] + y2c[...]) * dv
    p2d = (zd0[...] + zd1[...] + y2d[...]) * dv
    h2 = jnp.maximum(
        jnp.dot(p2a, w2a[...], preferred_element_type=jnp.float32)
        + jnp.dot(p2b, w2b[...], preferred_element_type=jnp.float32)
        + jnp.dot(p2c, w2c[...], preferred_element_type=jnp.float32)
        + jnp.dot(p2d, w2d[...], preferred_element_type=jnp.float32)
        + b2[...], 0.0)
    t = jnp.dot(h2, w3[...], preferred_element_type=jnp.float32)
    y3 = t * dv
    y3a[...] = y3[:, :8]
    y3b[...] = y3[:, 8:]


def _stage_d_body(za0, za1, zb0, zb1, y3a, y3b, dinv, b3, out):
    zza = za0[...] + za1[...] + y3a[...]
    zzb = zb0[...] + zb1[...] + y3b[...]
    zz = jnp.concatenate([zza, zzb], axis=1)
    out[...] = zz * dinv[...] + b3[...]


_stage_a = pl.pallas_call(
    _stage_a_body,
    grid=(_GRID,),
    in_specs=[_row_spec(4), _row_spec(4), _row_spec(4)],
    out_specs=[_row_spec(1), _row_spec(4)],
    out_shape=[_f32((N, 1)), _f32((N, 4))],
)

_stage_b = pl.pallas_call(
    _stage_b_body,
    grid=(_GRID,),
    in_specs=[_row_spec(4), _row_spec(4), _row_spec(4), _row_spec(1),
              _full_spec(4, 32), _full_spec(1, 32)],
    out_specs=[_row_spec(8)] * 4,
    out_shape=[_f32((N, 8))] * 4,
)

_stage_c = pl.pallas_call(
    _stage_c_body,
    grid=(_GRID,),
    in_specs=[_row_spec(8)] * 8 + [_row_spec(8)] * 4 + [_row_spec(1)]
             + [_full_spec(8, 64)] * 4 + [_full_spec(64, 16),
                _full_spec(1, 64)],
    out_specs=[_row_spec(8)] * 2,
    out_shape=[_f32((N, 8))] * 2,
)

_stage_d = pl.pallas_call(
    _stage_d_body,
    grid=(_GRID,),
    in_specs=[_row_spec(8)] * 4 + [_row_spec(8)] * 2 + [_row_spec(1),
              _full_spec(1, 16)],
    out_specs=_row_spec(16),
    out_shape=_f32((N, 16)),
)


def kernel(x, edge_index, W1, b1, W2, b2, W3, b3):
    e32 = edge_index.astype(jnp.int32)
    src = e32[0]
    dst = e32[1]
    ones4 = jnp.ones((EB, 4), jnp.float32)
    zeros4 = jnp.zeros((RPT, 4), jnp.float32)
    zeros8 = jnp.zeros((RPT, 8), jnp.float32)

    degp = _deg_kernel(dst, ones4, zeros4)
    dinv, y1 = _stage_a(degp[:N], degp[NP:NP + N], x)

    z1p, = _prop4(src, dst, y1, zeros4)
    y2a, y2b, y2c, y2d = _stage_b(z1p[:N], z1p[NP:NP + N], y1, dinv,
                                  W1, b1.reshape(1, 32))

    z2a, z2b, z2c, z2d = _prop8x4(src, dst, y2a, y2b, y2c, y2d, zeros8)
    y3a, y3b = _stage_c(
        z2a[:N], z2a[NP:NP + N], z2b[:N], z2b[NP:NP + N],
        z2c[:N], z2c[NP:NP + N], z2d[:N], z2d[NP:NP + N],
        y2a, y2b, y2c, y2d, dinv,
        W2[0:8], W2[8:16], W2[16:24], W2[24:32], W3, b2.reshape(1, 64))

    z3a, z3b = _prop8x2(src, dst, y3a, y3b, zeros8)
    out = _stage_d(z3a[:N], z3a[NP:NP + N], z3b[:N], z3b[NP:NP + N],
                   y3a, y3b, dinv, b3.reshape(1, 16))
    return out


# speedup vs baseline: 16.8276x; 3.1470x over previous
import functools

import jax
import jax.numpy as jnp
from jax import lax
from jax.experimental import pallas as pl
from jax.experimental.pallas import tpu as pltpu
from jax.experimental.pallas import tpu_sc as plsc

N = 100000
E = 1600000

NC = 2
NS = 16
NW = NC * NS
EPW = E // NW
EB = 2000
NCHUNK = EPW // EB
NP = 100096
RPT = NP // NS

_mesh = plsc.VectorSubcoreMesh(core_axis_name="c", subcore_axis_name="s")


def _f32(shape):
    return jax.ShapeDtypeStruct(shape, jnp.float32)


def _make_deg_kernel():
    @functools.partial(
        pl.kernel,
        out_type=_f32((NC * NP, 4)),
        mesh=_mesh,
        compiler_params=pltpu.CompilerParams(use_tc_tiling_on_sc=False),
        scratch_types=[
            pltpu.VMEM((EB,), jnp.int32),
            pltpu.VMEM((EB, 4), jnp.float32),
            pltpu.VMEM((RPT, 4), jnp.float32),
            pltpu.VMEM_SHARED((NP, 4), jnp.float32),
        ],
    )
    def deg_kernel(dst_hbm, ones_hbm, zeros_hbm, out_hbm, didx, ones_v, wbuf, zsh):
        c = lax.axis_index("c")
        s = lax.axis_index("s")
        wid = s * NC + c
        pltpu.sync_copy(zeros_hbm, wbuf)
        pltpu.sync_copy(wbuf, zsh.at[pl.ds(s * RPT, RPT)])
        pltpu.sync_copy(ones_hbm, ones_v)
        plsc.subcore_barrier()
        base0 = wid * EPW

        def step(i, carry):
            base = base0 + i * EB
            pltpu.sync_copy(dst_hbm.at[pl.ds(base, EB)], didx)
            pltpu.sync_copy(ones_v, zsh.at[didx], add=True)
            return carry

        lax.fori_loop(0, NCHUNK, step, 0)
        plsc.subcore_barrier()
        pltpu.sync_copy(zsh.at[pl.ds(s * RPT, RPT)], wbuf)
        pltpu.sync_copy(wbuf, out_hbm.at[pl.ds(c * NP + s * RPT, RPT)])

    return deg_kernel


def _make_prop_kernel(fc, n_tables):
    @functools.partial(
        pl.kernel,
        out_type=[_f32((NC * NP, fc)) for _ in range(n_tables)],
        mesh=_mesh,
        compiler_params=pltpu.CompilerParams(use_tc_tiling_on_sc=False),
        scratch_types=[
            pltpu.VMEM((EB,), jnp.int32),
            pltpu.VMEM((EB,), jnp.int32),
            pltpu.VMEM((EB, fc), jnp.float32),
            pltpu.VMEM((RPT, fc), jnp.float32),
            pltpu.VMEM_SHARED((NP, fc), jnp.float32),
            pltpu.SemaphoreType.DMA,
        ],
    )
    def prop_kernel(src_hbm, dst_hbm, *rest):
        tabs = rest[:n_tables]
        zeros_hbm = rest[n_tables]
        outs = rest[n_tables + 1:2 * n_tables + 1]
        sidx, didx, rows, wbuf, zsh, gsem = rest[2 * n_tables + 1:]
        c = lax.axis_index("c")
        s = lax.axis_index("s")
        wid = s * NC + c
        base0 = wid * EPW
        for t in range(n_tables):
            pltpu.sync_copy(zeros_hbm, wbuf)
            pltpu.sync_copy(wbuf, zsh.at[pl.ds(s * RPT, RPT)])
            plsc.subcore_barrier()

            def step(i, carry):
                base = base0 + i * EB
                pltpu.sync_copy(src_hbm.at[pl.ds(base, EB)], sidx)
                pltpu.sync_copy(dst_hbm.at[pl.ds(base, EB)], didx)
                pltpu.async_copy(tabs[t].at[sidx], rows, gsem).wait()
                pltpu.sync_copy(rows, zsh.at[didx], add=True)
                return carry

            lax.fori_loop(0, NCHUNK, step, 0)
            plsc.subcore_barrier()
            pltpu.sync_copy(zsh.at[pl.ds(s * RPT, RPT)], wbuf)
            pltpu.sync_copy(wbuf, outs[t].at[pl.ds(c * NP + s * RPT, RPT)])
            if t + 1 < n_tables:
                plsc.subcore_barrier()

    return prop_kernel


_deg_kernel = _make_deg_kernel()
_prop4 = _make_prop_kernel(4, 1)
_prop8x4 = _make_prop_kernel(8, 4)
_prop8x2 = _make_prop_kernel(8, 2)


_BLK = 2000
_GRID = N // _BLK


def _row_spec(f):
    return pl.BlockSpec((_BLK, f), lambda i: (i, 0))


def _full_spec(a, b):
    return pl.BlockSpec((a, b), lambda i: (0, 0))


def _stage_a_body(d0, d1, x, dinv, y1):
    deg = d0[:, 0:1] + d1[:, 0:1] + 1.0
    dv = lax.rsqrt(deg)
    dinv[...] = dv
    y1[...] = x[...] * dv


def _stage_b_body(z0, z1, y1, dinv, w1, b1, y2a, y2b, y2c, y2d):
    zz = z0[...] + z1[...] + y1[...]
    p1 = zz * dinv[...]
    h1 = jnp.maximum(jnp.dot(p1, w1[...], preferred_element_type=jnp.float32)
                     + b1[...], 0.0)
    y2 = h1 * dinv[...]
    y2a[...] = y2[:, 0:8]
    y2b[...] = y2[:, 8:16]
    y2c[...] = y2[:, 16:24]
    y2d[...] = y2[:, 24:32]


def _stage_c_body(za0, za1, zb0, zb1, zc0, zc1, zd0, zd1,
                  y2a, y2b, y2c, y2d, dinv, w2a, w2b, w2c, w2d, w3, b2,
                  y3a, y3b):
    dv = dinv[...]
    p2a = (za0[...] + za1[...] + y2a[...]) * dv
    p2b = (zb0[...] + zb1[...] + y2b[...]) * dv
    p2c = (zc0[...] + zc1[...] + y2c[...]) * dv
    p2d = (zd0[...] + zd1[...] + y2d[...]) * dv
    h2 = jnp.maximum(
        jnp.dot(p2a, w2a[...], preferred_element_type=jnp.float32)
        + jnp.dot(p2b, w2b[...], preferred_element_type=jnp.float32)
        + jnp.dot(p2c, w2c[...], preferred_element_type=jnp.float32)
        + jnp.dot(p2d, w2d[...], preferred_element_type=jnp.float32)
        + b2[...], 0.0)
    t = jnp.dot(h2, w3[...], preferred_element_type=jnp.float32)
    y3 = t * dv
    y3a[...] = y3[:, :8]
    y3b[...] = y3[:, 8:]


def _stage_d_body(za0, za1, zb0, zb1, y3a, y3b, dinv, b3, out):
    zza = za0[...] + za1[...] + y3a[...]
    zzb = zb0[...] + zb1[...] + y3b[...]
    zz = jnp.concatenate([zza, zzb], axis=1)
    out[...] = zz * dinv[...] + b3[...]


_stage_a = pl.pallas_call(
    _stage_a_body,
    grid=(_GRID,),
    in_specs=[_row_spec(4), _row_spec(4), _row_spec(4)],
    out_specs=[_row_spec(1), _row_spec(4)],
    out_shape=[_f32((N, 1)), _f32((N, 4))],
)

_stage_b = pl.pallas_call(
    _stage_b_body,
    grid=(_GRID,),
    in_specs=[_row_spec(4), _row_spec(4), _row_spec(4), _row_spec(1),
              _full_spec(4, 32), _full_spec(1, 32)],
    out_specs=[_row_spec(8)] * 4,
    out_shape=[_f32((N, 8))] * 4,
)

_stage_c = pl.pallas_call(
    _stage_c_body,
    grid=(_GRID,),
    in_specs=[_row_spec(8)] * 8 + [_row_spec(8)] * 4 + [_row_spec(1)]
             + [_full_spec(8, 64)] * 4 + [_full_spec(64, 16),
                _full_spec(1, 64)],
    out_specs=[_row_spec(8)] * 2,
    out_shape=[_f32((N, 8))] * 2,
)

_stage_d = pl.pallas_call(
    _stage_d_body,
    grid=(_GRID,),
    in_specs=[_row_spec(8)] * 4 + [_row_spec(8)] * 2 + [_row_spec(1),
              _full_spec(1, 16)],
    out_specs=_row_spec(16),
    out_shape=_f32((N, 16)),
)


def kernel(x, edge_index, W1, b1, W2, b2, W3, b3):
    e32 = edge_index.astype(jnp.int32)
    src = e32[0]
    dst = e32[1]
    ones4 = jnp.ones((EB, 4), jnp.float32)
    zeros4 = jnp.zeros((RPT, 4), jnp.float32)
    zeros8 = jnp.zeros((RPT, 8), jnp.float32)

    degp = _deg_kernel(dst, ones4, zeros4)
    dinv, y1 = _stage_a(degp[:N], degp[NP:NP + N], x)

    z1p, = _prop4(src, dst, y1, zeros4)
    y2a, y2b, y2c, y2d = _stage_b(z1p[:N], z1p[NP:NP + N], y1, dinv,
                                  W1, b1.reshape(1, 32))

    z2a, z2b, z2c, z2d = _prop8x4(src, dst, y2a, y2b, y2c, y2d, zeros8)
    y3a, y3b = _stage_c(
        z2a[:N], z2a[NP:NP + N], z2b[:N], z2b[NP:NP + N],
        z2c[:N], z2c[NP:NP + N], z2d[:N], z2d[NP:NP + N],
        y2a, y2b, y2c, y2d, dinv,
        W2[0:8], W2[8:16], W2[16:24], W2[24:32], W3, b2.reshape(1, 64))

    z3a, z3b = _prop8x2(src, dst, y3a, y3b, zeros8)
    out = _stage_d(z3a[:N], z3a[NP:NP + N], z3b[:N], z3b[NP:NP + N],
                   y3a, y3b, dinv, b3.reshape(1, 16))
    return out


# speedup vs baseline: 18.8741x; 1.1216x over previous
import functools

import jax
import jax.numpy as jnp
from jax import lax
from jax.experimental import pallas as pl
from jax.experimental.pallas import tpu as pltpu
from jax.experimental.pallas import tpu_sc as plsc

N = 100000
E = 1600000

NC = 2
NS = 16
NW = NC * NS
EPW = E // NW
EB = 2000
NCHUNK = EPW // EB
NP = 100096
RPT = NP // NS

_mesh = plsc.VectorSubcoreMesh(core_axis_name="c", subcore_axis_name="s")


def _f32(shape):
    return jax.ShapeDtypeStruct(shape, jnp.float32)


def _make_deg_kernel():
    @functools.partial(
        pl.kernel,
        out_type=_f32((NC * NP, 4)),
        mesh=_mesh,
        compiler_params=pltpu.CompilerParams(use_tc_tiling_on_sc=False),
        scratch_types=[
            pltpu.VMEM((EB,), jnp.int32),
            pltpu.VMEM((EB, 4), jnp.float32),
            pltpu.VMEM((RPT, 4), jnp.float32),
            pltpu.VMEM_SHARED((NP, 4), jnp.float32),
        ],
    )
    def deg_kernel(dst_hbm, ones_hbm, zeros_hbm, out_hbm, didx, ones_v, wbuf, zsh):
        c = lax.axis_index("c")
        s = lax.axis_index("s")
        wid = s * NC + c
        pltpu.sync_copy(zeros_hbm, wbuf)
        pltpu.sync_copy(wbuf, zsh.at[pl.ds(s * RPT, RPT)])
        pltpu.sync_copy(ones_hbm, ones_v)
        plsc.subcore_barrier()
        base0 = wid * EPW

        def step(i, carry):
            base = base0 + i * EB
            pltpu.sync_copy(dst_hbm.at[pl.ds(base, EB)], didx)
            pltpu.sync_copy(ones_v, zsh.at[didx], add=True)
            return carry

        lax.fori_loop(0, NCHUNK, step, 0)
        plsc.subcore_barrier()
        pltpu.sync_copy(zsh.at[pl.ds(s * RPT, RPT)], wbuf)
        pltpu.sync_copy(wbuf, out_hbm.at[pl.ds(c * NP + s * RPT, RPT)])

    return deg_kernel


def _make_prop_kernel(fc, n_tables):
    @functools.partial(
        pl.kernel,
        out_type=[_f32((NC * NP, fc)) for _ in range(n_tables)],
        mesh=_mesh,
        compiler_params=pltpu.CompilerParams(use_tc_tiling_on_sc=False),
        scratch_types=[
            pltpu.VMEM((EB,), jnp.int32),
            pltpu.VMEM((EB,), jnp.int32),
            pltpu.VMEM((EB, fc), jnp.float32),
            pltpu.VMEM((RPT, fc), jnp.float32),
            pltpu.VMEM_SHARED((NP, fc), jnp.float32),
            pltpu.SemaphoreType.DMA,
        ],
    )
    def prop_kernel(src_hbm, dst_hbm, *rest):
        tabs = rest[:n_tables]
        zeros_hbm = rest[n_tables]
        outs = rest[n_tables + 1:2 * n_tables + 1]
        sidx, didx, rows, wbuf, zsh, gsem = rest[2 * n_tables + 1:]
        c = lax.axis_index("c")
        s = lax.axis_index("s")
        wid = s * NC + c
        base0 = wid * EPW
        for t in range(n_tables):
            pltpu.sync_copy(zeros_hbm, wbuf)
            pltpu.sync_copy(wbuf, zsh.at[pl.ds(s * RPT, RPT)])
            plsc.subcore_barrier()

            def step(i, carry):
                base = base0 + i * EB
                pltpu.sync_copy(src_hbm.at[pl.ds(base, EB)], sidx)
                pltpu.sync_copy(dst_hbm.at[pl.ds(base, EB)], didx)
                pltpu.async_copy(tabs[t].at[sidx], rows, gsem).wait()
                pltpu.sync_copy(rows, zsh.at[didx], add=True)
                return carry

            lax.fori_loop(0, NCHUNK, step, 0)
            plsc.subcore_barrier()
            pltpu.sync_copy(zsh.at[pl.ds(s * RPT, RPT)], wbuf)
            pltpu.sync_copy(wbuf, outs[t].at[pl.ds(c * NP + s * RPT, RPT)])
            if t + 1 < n_tables:
                plsc.subcore_barrier()

    return prop_kernel


_deg_kernel = _make_deg_kernel()
_prop4 = _make_prop_kernel(4, 1)
_prop8x4 = _make_prop_kernel(8, 4)
_prop8x2 = _make_prop_kernel(8, 2)


import numpy as np

_GRID = 23
_R4 = NP * 4 // 128
_R8 = NP * 8 // 128
_B4 = _R4 // _GRID
_B8 = _R8 // _GRID


def _np_e8():
    e = np.zeros((2, 128, 128), np.float32)
    for k in range(2):
        for q in range(16):
            for i in range(8):
                e[k, (16 * k + q) * 4, q * 8 + i] = 1.0
    return e


_E8 = _np_e8()

_D1 = np.zeros((2, 32, 16), np.float32)
for _k in range(2):
    for _q in range(16):
        _D1[_k, 16 * _k + _q, _q] = 1.0
_D2 = np.zeros((8, 16, 2), np.float32)
for _k in range(8):
    for _m in range(2):
        _D2[_k, 2 * _k + _m, _m] = 1.0
_EYE16 = np.eye(16, dtype=np.float32)
_PA = np.einsum('qv,ij->qivj', _EYE16, np.eye(8, 16, dtype=np.float32)
                ).reshape(128, 256)
_PB = np.einsum('qv,ij->qivj', _EYE16,
                np.concatenate([np.zeros((8, 8), np.float32),
                                np.eye(8, dtype=np.float32)], 1)
                ).reshape(128, 256)
def _pack_weights(W1, b1, W2, b2, W3, b3):
    w1b = jnp.stack([
        jnp.einsum('pq,fi->pfqi', _D1[k], W1[:, 8 * t:8 * t + 8]
                   ).reshape(128, 128)
        for t in range(4) for k in range(2)])
    b1p = jnp.tile(b1.reshape(4, 8), (1, 32))
    w2b = jnp.stack([
        jnp.einsum('qm,ig->qimg', _D2[k], W2[8 * t:8 * t + 8, :]
                   ).reshape(128, 128)
        for t in range(4) for k in range(8)])
    b2p = jnp.tile(b2, 2).reshape(1, 128)
    wy3a = jnp.einsum('uq,gi->ugqi', _EYE16, W3[:, :8]).reshape(1024, 128)
    wy3b = jnp.einsum('uq,gi->ugqi', _EYE16, W3[:, 8:]).reshape(1024, 128)
    b3p = jnp.tile(b3, 16).reshape(1, 256)
    return w1b, b1p, w2b, b2p, wy3a, wy3b, b3p


def _spec(rows, lanes):
    return pl.BlockSpec((rows, lanes), lambda i: (i, 0))


def _const(shape):
    nd = len(shape)
    return pl.BlockSpec(shape, lambda i: (0,) * nd)


def _dotf(a, b):
    return jnp.dot(a, b, preferred_element_type=jnp.float32)


def _stage_a_body(d0, d1, xp, e, dinv4, dinv8w, y1):
    deg = d0[...] + d1[...] + 1.0
    dv = lax.rsqrt(deg)
    dinv4[...] = dv
    dinv8w[...] = jnp.concatenate([_dotf(dv, e[0]), _dotf(dv, e[1])], axis=1)
    y1[...] = xp[...] * dv


def _stage_b_body(z0, z1, y1, dinv4, dinv8w, w, b, y2a, y2b, y2c, y2d):
    p1 = (z0[...] + z1[...] + y1[...]) * dinv4[...]
    dw = dinv8w[...]
    for t, ref in enumerate([y2a, y2b, y2c, y2d]):
        h = jnp.concatenate([_dotf(p1, w[2 * t]), _dotf(p1, w[2 * t + 1])],
                            axis=1)
        ref[...] = jnp.maximum(h + b[t], 0.0) * dw


def _stage_c_body(za0, za1, zb0, zb1, zc0, zc1, zd0, zd1,
                  y2a, y2b, y2c, y2d, dinv8, w2, wy3a, wy3b, b2p,
                  y3a, y3b):
    dv = dinv8[...]
    ps = [(za0[...] + za1[...] + y2a[...]) * dv,
          (zb0[...] + zb1[...] + y2b[...]) * dv,
          (zc0[...] + zc1[...] + y2c[...]) * dv,
          (zd0[...] + zd1[...] + y2d[...]) * dv]
    hs = []
    for k in range(8):
        acc = _dotf(ps[0], w2[k])
        for t in range(1, 4):
            acc = acc + _dotf(ps[t], w2[t * 8 + k])
        hs.append(jnp.maximum(acc + b2p[...], 0.0))
    h2w = jnp.concatenate(hs, axis=1)
    y3a[...] = _dotf(h2w, wy3a[...]) * dv
    y3b[...] = _dotf(h2w, wy3b[...]) * dv


def _stage_d_body(za0, za1, zb0, zb1, y3a, y3b, dinv8, pa, pb, b3p, outw):
    dv = dinv8[...]
    fa = (za0[...] + za1[...] + y3a[...]) * dv
    fb = (zb0[...] + zb1[...] + y3b[...]) * dv
    outw[...] = _dotf(fa, pa[...]) + _dotf(fb, pb[...]) + b3p[...]


_stage_a = pl.pallas_call(
    _stage_a_body,
    grid=(_GRID,),
    in_specs=[_spec(_B4, 128)] * 3 + [_const((2, 128, 128))],
    out_specs=[_spec(_B4, 128), _spec(_B4, 256), _spec(_B4, 128)],
    out_shape=[_f32((_R4, 128)), _f32((_R4, 256)), _f32((_R4, 128))],
)

_stage_b = pl.pallas_call(
    _stage_b_body,
    grid=(_GRID,),
    in_specs=[_spec(_B4, 128)] * 4 + [_spec(_B4, 256),
              _const((8, 128, 128)), _const((4, 256))],
    out_specs=[_spec(_B4, 256)] * 4,
    out_shape=[_f32((_R4, 256))] * 4,
)

_stage_c = pl.pallas_call(
    _stage_c_body,
    grid=(_GRID,),
    in_specs=[_spec(_B8, 128)] * 13 + [_const((32, 128, 128)),
              _const((1024, 128)), _const((1024, 128)), _const((1, 128))],
    out_specs=[_spec(_B8, 128)] * 2,
    out_shape=[_f32((_R8, 128))] * 2,
)

_stage_d = pl.pallas_call(
    _stage_d_body,
    grid=(_GRID,),
    in_specs=[_spec(_B8, 128)] * 7 + [_const((128, 256)), _const((128, 256)),
              _const((1, 256))],
    out_specs=_spec(_B8, 256),
    out_shape=_f32((_R8, 256)),
)


def kernel(x, edge_index, W1, b1, W2, b2, W3, b3):
    e32 = edge_index.astype(jnp.int32)
    src = e32[0]
    dst = e32[1]
    ones4 = jnp.ones((EB, 4), jnp.float32)
    zeros4 = jnp.zeros((RPT, 4), jnp.float32)
    zeros8 = jnp.zeros((RPT, 8), jnp.float32)
    w1b, b1p, w2b, b2p, wy3a, wy3b, b3p = _pack_weights(W1, b1, W2, b2, W3, b3)
    xp = jnp.pad(x, ((0, NP - N), (0, 0))).reshape(_R4, 128)

    def halves(a, fc):
        f = a.reshape(-1)
        return (f[:NP * fc].reshape(NP * fc // 128, 128),
                f[NP * fc:].reshape(NP * fc // 128, 128))

    degp = _deg_kernel(dst, ones4, zeros4)
    d0, d1 = halves(degp, 4)
    dinv4, dinv8w, y1 = _stage_a(d0, d1, xp, jnp.asarray(_E8))

    z1p, = _prop4(src, dst, y1.reshape(NP, 4), zeros4)
    z10, z11 = halves(z1p, 4)
    y2 = _stage_b(z10, z11, y1, dinv4, dinv8w, w1b, b1p)

    z2 = _prop8x4(src, dst, *[t.reshape(NP, 8) for t in y2], zeros8)
    dinv8 = dinv8w.reshape(_R8, 128)
    z2h = [h for t in z2 for h in halves(t, 8)]
    y2v = [t.reshape(_R8, 128) for t in y2]
    y3a, y3b = _stage_c(*z2h, *y2v, dinv8, w2b, wy3a, wy3b, b2p)

    z3a, z3b = _prop8x2(src, dst, y3a.reshape(NP, 8), y3b.reshape(NP, 8),
                        zeros8)
    za0, za1 = halves(z3a, 8)
    zb0, zb1 = halves(z3b, 8)
    outw = _stage_d(za0, za1, zb0, zb1, y3a, y3b, dinv8,
                    jnp.asarray(_PA), jnp.asarray(_PB), b3p)
    return outw.reshape(NP, 16)[:N]


# speedup vs baseline: 21.0083x; 1.1131x over previous
import functools

import jax
import jax.numpy as jnp
from jax import lax
from jax.experimental import pallas as pl
from jax.experimental.pallas import tpu as pltpu
from jax.experimental.pallas import tpu_sc as plsc

N = 100000
E = 1600000

NC = 2
NS = 16
NW = NC * NS
EPW = E // NW
EB = 1000
NCHUNK = EPW // EB
NP = 100096
RPT = NP // NS

_mesh = plsc.VectorSubcoreMesh(core_axis_name="c", subcore_axis_name="s")


def _f32(shape):
    return jax.ShapeDtypeStruct(shape, jnp.float32)


def _make_deg_kernel():
    @functools.partial(
        pl.kernel,
        out_type=_f32((NC * NP, 4)),
        mesh=_mesh,
        compiler_params=pltpu.CompilerParams(use_tc_tiling_on_sc=False),
        scratch_types=[
            pltpu.VMEM((EB,), jnp.int32),
            pltpu.VMEM((EB,), jnp.int32),
            pltpu.VMEM((EB, 4), jnp.float32),
            pltpu.VMEM((RPT, 4), jnp.float32),
            pltpu.VMEM_SHARED((NP, 4), jnp.float32),
            pltpu.SemaphoreType.DMA,
            pltpu.SemaphoreType.DMA,
            pltpu.SemaphoreType.DMA,
            pltpu.SemaphoreType.DMA,
        ],
    )
    def deg_kernel(e_hbm, ones_hbm, zeros_hbm, out_hbm, didx0, didx1,
                   ones_v, wbuf, zsh, is0, is1, ss0, ss1):
        c = lax.axis_index("c")
        s = lax.axis_index("s")
        wid = s * NC + c
        pltpu.sync_copy(zeros_hbm, wbuf)
        pltpu.sync_copy(wbuf, zsh.at[pl.ds(s * RPT, RPT)])
        pltpu.sync_copy(ones_hbm, ones_v)
        plsc.subcore_barrier()
        base0 = wid * EPW

        def step(i, carry):
            base = base0 + 2 * i * EB
            i0 = pltpu.async_copy(e_hbm.at[1, pl.ds(base, EB)], didx0, is0)
            i1 = pltpu.async_copy(e_hbm.at[1, pl.ds(base + EB, EB)], didx1, is1)
            i0.wait()
            s0 = pltpu.async_copy(ones_v, zsh.at[didx0], ss0, add=True)
            i1.wait()
            s1 = pltpu.async_copy(ones_v, zsh.at[didx1], ss1, add=True)
            s0.wait()
            s1.wait()
            return carry

        lax.fori_loop(0, NCHUNK // 2, step, 0)
        plsc.subcore_barrier()
        pltpu.sync_copy(zsh.at[pl.ds(s * RPT, RPT)], wbuf)
        pltpu.sync_copy(wbuf, out_hbm.at[pl.ds(c * NP + s * RPT, RPT)])

    return deg_kernel


def _make_prop_kernel(fc, n_tables):
    @functools.partial(
        pl.kernel,
        out_type=[_f32((NC * NP, fc)) for _ in range(n_tables)],
        mesh=_mesh,
        compiler_params=pltpu.CompilerParams(use_tc_tiling_on_sc=False),
        scratch_types=(
            [pltpu.VMEM((EB,), jnp.int32)] * 4 +
            [pltpu.VMEM((EB, fc), jnp.float32)] * 2 +
            [pltpu.VMEM((RPT, fc), jnp.float32),
             pltpu.VMEM_SHARED((NP, fc), jnp.float32)] +
            [pltpu.SemaphoreType.DMA] * 8
        ),
    )
    def prop_kernel(e_hbm, *rest):
        tabs = rest[:n_tables]
        zeros_hbm = rest[n_tables]
        outs = rest[n_tables + 1:2 * n_tables + 1]
        (sidx0, sidx1, didx0, didx1, rows0, rows1, wbuf, zsh,
         ia0, ia1, ib0, ib1, gs0, gs1, ss0, ss1) = rest[2 * n_tables + 1:]
        c = lax.axis_index("c")
        s = lax.axis_index("s")
        wid = s * NC + c
        base0 = wid * EPW
        for t in range(n_tables):
            pltpu.sync_copy(zeros_hbm, wbuf)
            pltpu.sync_copy(wbuf, zsh.at[pl.ds(s * RPT, RPT)])
            plsc.subcore_barrier()

            def step(i, carry):
                base = base0 + 2 * i * EB
                da0 = pltpu.async_copy(e_hbm.at[0, pl.ds(base, EB)], sidx0, ia0)
                db0 = pltpu.async_copy(e_hbm.at[1, pl.ds(base, EB)], didx0, ib0)
                da1 = pltpu.async_copy(e_hbm.at[0, pl.ds(base + EB, EB)],
                                       sidx1, ia1)
                db1 = pltpu.async_copy(e_hbm.at[1, pl.ds(base + EB, EB)],
                                       didx1, ib1)
                da0.wait()
                g0 = pltpu.async_copy(tabs[t].at[sidx0], rows0, gs0)
                da1.wait()
                g1 = pltpu.async_copy(tabs[t].at[sidx1], rows1, gs1)
                g0.wait()
                db0.wait()
                s0 = pltpu.async_copy(rows0, zsh.at[didx0], ss0, add=True)
                g1.wait()
                db1.wait()
                s1 = pltpu.async_copy(rows1, zsh.at[didx1], ss1, add=True)
                s0.wait()
                s1.wait()
                return carry

            lax.fori_loop(0, NCHUNK // 2, step, 0)
            plsc.subcore_barrier()
            pltpu.sync_copy(zsh.at[pl.ds(s * RPT, RPT)], wbuf)
            pltpu.sync_copy(wbuf, outs[t].at[pl.ds(c * NP + s * RPT, RPT)])
            if t + 1 < n_tables:
                plsc.subcore_barrier()

    return prop_kernel


_deg_kernel = _make_deg_kernel()
_prop4 = _make_prop_kernel(4, 1)
_prop8x4 = _make_prop_kernel(8, 4)
_prop8x2 = _make_prop_kernel(8, 2)


import numpy as np

_GRID = 23
_R4 = NP * 4 // 128
_R8 = NP * 8 // 128
_B4 = _R4 // _GRID
_B8 = _R8 // _GRID


def _np_e8():
    e = np.zeros((2, 128, 128), np.float32)
    for k in range(2):
        for q in range(16):
            for i in range(8):
                e[k, (16 * k + q) * 4, q * 8 + i] = 1.0
    return e


_E8 = _np_e8()

_D1 = np.zeros((2, 32, 16), np.float32)
for _k in range(2):
    for _q in range(16):
        _D1[_k, 16 * _k + _q, _q] = 1.0
_D2 = np.zeros((8, 16, 2), np.float32)
for _k in range(8):
    for _m in range(2):
        _D2[_k, 2 * _k + _m, _m] = 1.0
_EYE16 = np.eye(16, dtype=np.float32)
_PA = np.einsum('qv,ij->qivj', _EYE16, np.eye(8, 16, dtype=np.float32)
                ).reshape(128, 256)
_PB = np.einsum('qv,ij->qivj', _EYE16,
                np.concatenate([np.zeros((8, 8), np.float32),
                                np.eye(8, dtype=np.float32)], 1)
                ).reshape(128, 256)
def _pack_weights(W1, b1, W2, b2, W3, b3):
    w1b = jnp.stack([
        jnp.einsum('pq,fi->pfqi', _D1[k], W1[:, 8 * t:8 * t + 8]
                   ).reshape(128, 128)
        for t in range(4) for k in range(2)])
    b1p = jnp.tile(b1.reshape(4, 8), (1, 32))
    w2b = jnp.stack([
        jnp.einsum('qm,ig->qimg', _D2[k], W2[8 * t:8 * t + 8, :]
                   ).reshape(128, 128)
        for t in range(4) for k in range(8)])
    b2p = jnp.tile(b2, 2).reshape(1, 128)
    wy3a = jnp.einsum('uq,gi->ugqi', _EYE16, W3[:, :8]).reshape(1024, 128)
    wy3b = jnp.einsum('uq,gi->ugqi', _EYE16, W3[:, 8:]).reshape(1024, 128)
    b3p = jnp.tile(b3, 16).reshape(1, 256)
    return w1b, b1p, w2b, b2p, wy3a, wy3b, b3p


def _spec(rows, lanes):
    return pl.BlockSpec((rows, lanes), lambda i: (i, 0))


def _const(shape):
    nd = len(shape)
    return pl.BlockSpec(shape, lambda i: (0,) * nd)


def _dotf(a, b):
    return jnp.dot(a, b, preferred_element_type=jnp.float32)


def _stage_a_body(d0, d1, xp, e, dinv4, dinv8w, y1):
    deg = d0[...] + d1[...] + 1.0
    dv = lax.rsqrt(deg)
    dinv4[...] = dv
    dinv8w[...] = jnp.concatenate([_dotf(dv, e[0]), _dotf(dv, e[1])], axis=1)
    y1[...] = xp[...] * dv


def _stage_b_body(z0, z1, y1, dinv4, dinv8w, w, b, y2a, y2b, y2c, y2d):
    p1 = (z0[...] + z1[...] + y1[...]) * dinv4[...]
    dw = dinv8w[...]
    for t, ref in enumerate([y2a, y2b, y2c, y2d]):
        h = jnp.concatenate([_dotf(p1, w[2 * t]), _dotf(p1, w[2 * t + 1])],
                            axis=1)
        ref[...] = jnp.maximum(h + b[t], 0.0) * dw


def _stage_c_body(za0, za1, zb0, zb1, zc0, zc1, zd0, zd1,
                  y2a, y2b, y2c, y2d, dinv8, w2, wy3a, wy3b, b2p,
                  y3a, y3b):
    dv = dinv8[...]
    ps = [(za0[...] + za1[...] + y2a[...]) * dv,
          (zb0[...] + zb1[...] + y2b[...]) * dv,
          (zc0[...] + zc1[...] + y2c[...]) * dv,
          (zd0[...] + zd1[...] + y2d[...]) * dv]
    hs = []
    for k in range(8):
        acc = _dotf(ps[0], w2[k])
        for t in range(1, 4):
            acc = acc + _dotf(ps[t], w2[t * 8 + k])
        hs.append(jnp.maximum(acc + b2p[...], 0.0))
    h2w = jnp.concatenate(hs, axis=1)
    y3a[...] = _dotf(h2w, wy3a[...]) * dv
    y3b[...] = _dotf(h2w, wy3b[...]) * dv


def _stage_d_body(za0, za1, zb0, zb1, y3a, y3b, dinv8, pa, pb, b3p, outw):
    dv = dinv8[...]
    fa = (za0[...] + za1[...] + y3a[...]) * dv
    fb = (zb0[...] + zb1[...] + y3b[...]) * dv
    outw[...] = _dotf(fa, pa[...]) + _dotf(fb, pb[...]) + b3p[...]


_stage_a = pl.pallas_call(
    _stage_a_body,
    grid=(_GRID,),
    in_specs=[_spec(_B4, 128)] * 3 + [_const((2, 128, 128))],
    out_specs=[_spec(_B4, 128), _spec(_B4, 256), _spec(_B4, 128)],
    out_shape=[_f32((_R4, 128)), _f32((_R4, 256)), _f32((_R4, 128))],
)

_stage_b = pl.pallas_call(
    _stage_b_body,
    grid=(_GRID,),
    in_specs=[_spec(_B4, 128)] * 4 + [_spec(_B4, 256),
              _const((8, 128, 128)), _const((4, 256))],
    out_specs=[_spec(_B4, 256)] * 4,
    out_shape=[_f32((_R4, 256))] * 4,
)

_stage_c = pl.pallas_call(
    _stage_c_body,
    grid=(_GRID,),
    in_specs=[_spec(_B8, 128)] * 13 + [_const((32, 128, 128)),
              _const((1024, 128)), _const((1024, 128)), _const((1, 128))],
    out_specs=[_spec(_B8, 128)] * 2,
    out_shape=[_f32((_R8, 128))] * 2,
)

_stage_d = pl.pallas_call(
    _stage_d_body,
    grid=(_GRID,),
    in_specs=[_spec(_B8, 128)] * 7 + [_const((128, 256)), _const((128, 256)),
              _const((1, 256))],
    out_specs=_spec(_B8, 256),
    out_shape=_f32((_R8, 256)),
)


def kernel(x, edge_index, W1, b1, W2, b2, W3, b3):
    e32 = edge_index.astype(jnp.int32)
    ones4 = jnp.ones((EB, 4), jnp.float32)
    zeros4 = jnp.zeros((RPT, 4), jnp.float32)
    zeros8 = jnp.zeros((RPT, 8), jnp.float32)
    w1b, b1p, w2b, b2p, wy3a, wy3b, b3p = _pack_weights(W1, b1, W2, b2, W3, b3)
    xp = jnp.pad(x, ((0, NP - N), (0, 0))).reshape(_R4, 128)

    def halves(a, fc):
        f = a.reshape(-1)
        return (f[:NP * fc].reshape(NP * fc // 128, 128),
                f[NP * fc:].reshape(NP * fc // 128, 128))

    degp = _deg_kernel(e32, ones4, zeros4)
    d0, d1 = halves(degp, 4)
    dinv4, dinv8w, y1 = _stage_a(d0, d1, xp, jnp.asarray(_E8))

    z1p, = _prop4(e32, y1.reshape(NP, 4), zeros4)
    z10, z11 = halves(z1p, 4)
    y2 = _stage_b(z10, z11, y1, dinv4, dinv8w, w1b, b1p)

    z2 = _prop8x4(e32, *[t.reshape(NP, 8) for t in y2], zeros8)
    dinv8 = dinv8w.reshape(_R8, 128)
    z2h = [h for t in z2 for h in halves(t, 8)]
    y2v = [t.reshape(_R8, 128) for t in y2]
    y3a, y3b = _stage_c(*z2h, *y2v, dinv8, w2b, wy3a, wy3b, b2p)

    z3a, z3b = _prop8x2(e32, y3a.reshape(NP, 8), y3b.reshape(NP, 8),
                        zeros8)
    za0, za1 = halves(z3a, 8)
    zb0, zb1 = halves(z3b, 8)
    outw = _stage_d(za0, za1, zb0, zb1, y3a, y3b, dinv8,
                    jnp.asarray(_PA), jnp.asarray(_PB), b3p)
    return outw.reshape(NP, 16)[:N]


# speedup vs baseline: 24.1747x; 1.1507x over previous
import functools

import jax
import jax.numpy as jnp
from jax import lax
from jax.experimental import pallas as pl
from jax.experimental.pallas import tpu as pltpu
from jax.experimental.pallas import tpu_sc as plsc

N = 100000
E = 1600000

NC = 2
NS = 16
NW = NC * NS
EPW = E // NW
EB = 1000
NCHUNK = EPW // EB
NP = 100096
RPT = NP // NS

_mesh = plsc.VectorSubcoreMesh(core_axis_name="c", subcore_axis_name="s")


def _f32(shape):
    return jax.ShapeDtypeStruct(shape, jnp.float32)


def _make_deg_kernel():
    @functools.partial(
        pl.kernel,
        out_type=_f32((NC * NP, 4)),
        mesh=_mesh,
        compiler_params=pltpu.CompilerParams(use_tc_tiling_on_sc=False),
        scratch_types=[
            pltpu.VMEM((EB,), jnp.int32),
            pltpu.VMEM((EB,), jnp.int32),
            pltpu.VMEM((EB, 4), jnp.float32),
            pltpu.VMEM((RPT, 4), jnp.float32),
            pltpu.VMEM_SHARED((NP, 4), jnp.float32),
            pltpu.SemaphoreType.DMA,
            pltpu.SemaphoreType.DMA,
            pltpu.SemaphoreType.DMA,
            pltpu.SemaphoreType.DMA,
        ],
    )
    def deg_kernel(e_hbm, ones_hbm, zeros_hbm, out_hbm, didx0, didx1,
                   ones_v, wbuf, zsh, is0, is1, ss0, ss1):
        c = lax.axis_index("c")
        s = lax.axis_index("s")
        wid = s * NC + c
        pltpu.sync_copy(zeros_hbm, wbuf)
        pltpu.sync_copy(wbuf, zsh.at[pl.ds(s * RPT, RPT)])
        pltpu.sync_copy(ones_hbm, ones_v)
        plsc.subcore_barrier()
        base0 = wid * EPW

        def step(i, carry):
            base = base0 + 2 * i * EB
            i0 = pltpu.async_copy(e_hbm.at[1, pl.ds(base, EB)], didx0, is0)
            i1 = pltpu.async_copy(e_hbm.at[1, pl.ds(base + EB, EB)], didx1, is1)
            i0.wait()
            s0 = pltpu.async_copy(ones_v, zsh.at[didx0], ss0, add=True)
            i1.wait()
            s1 = pltpu.async_copy(ones_v, zsh.at[didx1], ss1, add=True)
            s0.wait()
            s1.wait()
            return carry

        lax.fori_loop(0, NCHUNK // 2, step, 0)
        plsc.subcore_barrier()
        pltpu.sync_copy(zsh.at[pl.ds(s * RPT, RPT)], wbuf)
        pltpu.sync_copy(wbuf, out_hbm.at[pl.ds(c * NP + s * RPT, RPT)])

    return deg_kernel


def _make_prop_kernel(fc, n_tables):
    @functools.partial(
        pl.kernel,
        out_type=[_f32((NC * NP, fc)) for _ in range(n_tables)],
        mesh=_mesh,
        compiler_params=pltpu.CompilerParams(use_tc_tiling_on_sc=False),
        scratch_types=(
            [pltpu.VMEM((EB,), jnp.int32)] * 4 +
            [pltpu.VMEM((EB, fc), jnp.float32)] * 2 +
            [pltpu.VMEM((RPT, fc), jnp.float32),
             pltpu.VMEM_SHARED((NP, fc), jnp.float32)] +
            [pltpu.SemaphoreType.DMA] * 8
        ),
    )
    def prop_kernel(e_hbm, *rest):
        tabs = rest[:n_tables]
        zeros_hbm = rest[n_tables]
        outs = rest[n_tables + 1:2 * n_tables + 1]
        (sidx0, sidx1, didx0, didx1, rows0, rows1, wbuf, zsh,
         ia0, ia1, ib0, ib1, gs0, gs1, ss0, ss1) = rest[2 * n_tables + 1:]
        c = lax.axis_index("c")
        s = lax.axis_index("s")
        wid = s * NC + c
        base0 = wid * EPW
        for t in range(n_tables):
            pltpu.sync_copy(zeros_hbm, wbuf)
            pltpu.sync_copy(wbuf, zsh.at[pl.ds(s * RPT, RPT)])
            plsc.subcore_barrier()

            def step(i, carry):
                base = base0 + 2 * i * EB
                da0 = pltpu.async_copy(e_hbm.at[0, pl.ds(base, EB)], sidx0, ia0)
                db0 = pltpu.async_copy(e_hbm.at[1, pl.ds(base, EB)], didx0, ib0)
                da1 = pltpu.async_copy(e_hbm.at[0, pl.ds(base + EB, EB)],
                                       sidx1, ia1)
                db1 = pltpu.async_copy(e_hbm.at[1, pl.ds(base + EB, EB)],
                                       didx1, ib1)
                da0.wait()
                g0 = pltpu.async_copy(tabs[t].at[sidx0], rows0, gs0)
                da1.wait()
                g1 = pltpu.async_copy(tabs[t].at[sidx1], rows1, gs1)
                g0.wait()
                db0.wait()
                s0 = pltpu.async_copy(rows0, zsh.at[didx0], ss0, add=True)
                g1.wait()
                db1.wait()
                s1 = pltpu.async_copy(rows1, zsh.at[didx1], ss1, add=True)
                s0.wait()
                s1.wait()
                return carry

            lax.fori_loop(0, NCHUNK // 2, step, 0)
            plsc.subcore_barrier()
            pltpu.sync_copy(zsh.at[pl.ds(s * RPT, RPT)], wbuf)
            pltpu.sync_copy(wbuf, outs[t].at[pl.ds(c * NP + s * RPT, RPT)])
            if t + 1 < n_tables:
                plsc.subcore_barrier()

    return prop_kernel


_deg_kernel = _make_deg_kernel()
_prop4 = _make_prop_kernel(4, 1)
_prop8 = _make_prop_kernel(8, 1)


import numpy as np

_GRID = 23
_R4 = NP * 4 // 128
_R8 = NP * 8 // 128
_B4 = _R4 // _GRID
_B8 = _R8 // _GRID


def _np_e8():
    e = np.zeros((2, 128, 128), np.float32)
    for k in range(2):
        for q in range(16):
            for i in range(8):
                e[k, (16 * k + q) * 4, q * 8 + i] = 1.0
    return e


_E8 = _np_e8()

_D1 = np.zeros((2, 32, 16), np.float32)
for _k in range(2):
    for _q in range(16):
        _D1[_k, 16 * _k + _q, _q] = 1.0
_D2 = np.zeros((8, 16, 2), np.float32)
for _k in range(8):
    for _m in range(2):
        _D2[_k, 2 * _k + _m, _m] = 1.0
_EYE16 = np.eye(16, dtype=np.float32)
_PA = np.einsum('qv,ij->qivj', _EYE16, np.eye(8, 16, dtype=np.float32)
                ).reshape(128, 256)
_PB = np.einsum('qv,ij->qivj', _EYE16,
                np.concatenate([np.zeros((8, 8), np.float32),
                                np.eye(8, dtype=np.float32)], 1)
                ).reshape(128, 256)
def _pack_weights(W1, b1, W2, b2, W3, b3):
    w1b = jnp.stack([
        jnp.einsum('pq,fi->pfqi', _D1[k], W1[:, 8 * t:8 * t + 8]
                   ).reshape(128, 128)
        for t in range(4) for k in range(2)])
    b1p = jnp.tile(b1.reshape(4, 8), (1, 32))
    w2b = jnp.stack([
        jnp.einsum('qm,ig->qimg', _D2[k], W2[8 * t:8 * t + 8, :]
                   ).reshape(128, 128)
        for t in range(4) for k in range(8)])
    b2p = jnp.tile(b2, 2).reshape(1, 128)
    wy3a = jnp.einsum('uq,gi->ugqi', _EYE16, W3[:, :8]).reshape(1024, 128)
    wy3b = jnp.einsum('uq,gi->ugqi', _EYE16, W3[:, 8:]).reshape(1024, 128)
    b3p = jnp.tile(b3, 16).reshape(1, 256)
    return w1b, b1p, w2b, b2p, wy3a, wy3b, b3p


def _spec(rows, lanes):
    return pl.BlockSpec((rows, lanes), lambda i: (i, 0))


def _const(shape):
    nd = len(shape)
    return pl.BlockSpec(shape, lambda i: (0,) * nd)


def _dotf(a, b):
    return jnp.dot(a, b, preferred_element_type=jnp.float32)


def _stage_a_body(d0, d1, xp, e, dinv4, dinv8w, y1):
    deg = d0[...] + d1[...] + 1.0
    dv = lax.rsqrt(deg)
    dinv4[...] = dv
    dinv8w[...] = jnp.concatenate([_dotf(dv, e[0]), _dotf(dv, e[1])], axis=1)
    y1[...] = xp[...] * dv


def _stage_b_body(z0, z1, y1, dinv4, dinv8w, w, b, y2a, y2b, y2c, y2d):
    p1 = (z0[...] + z1[...] + y1[...]) * dinv4[...]
    dw = dinv8w[...]
    for t, ref in enumerate([y2a, y2b, y2c, y2d]):
        h = jnp.concatenate([_dotf(p1, w[2 * t]), _dotf(p1, w[2 * t + 1])],
                            axis=1)
        ref[...] = jnp.maximum(h + b[t], 0.0) * dw


def _stage_c_body(za0, za1, zb0, zb1, zc0, zc1, zd0, zd1,
                  y2a, y2b, y2c, y2d, dinv8, w2, wy3a, wy3b, b2p,
                  y3a, y3b):
    dv = dinv8[...]
    ps = [(za0[...] + za1[...] + y2a[...]) * dv,
          (zb0[...] + zb1[...] + y2b[...]) * dv,
          (zc0[...] + zc1[...] + y2c[...]) * dv,
          (zd0[...] + zd1[...] + y2d[...]) * dv]
    hs = []
    for k in range(8):
        acc = _dotf(ps[0], w2[k])
        for t in range(1, 4):
            acc = acc + _dotf(ps[t], w2[t * 8 + k])
        hs.append(jnp.maximum(acc + b2p[...], 0.0))
    h2w = jnp.concatenate(hs, axis=1)
    y3a[...] = _dotf(h2w, wy3a[...]) * dv
    y3b[...] = _dotf(h2w, wy3b[...]) * dv


def _stage_d_body(za0, za1, zb0, zb1, y3a, y3b, dinv8, pa, pb, b3p, outw):
    dv = dinv8[...]
    fa = (za0[...] + za1[...] + y3a[...]) * dv
    fb = (zb0[...] + zb1[...] + y3b[...]) * dv
    outw[...] = _dotf(fa, pa[...]) + _dotf(fb, pb[...]) + b3p[...]


_stage_a = pl.pallas_call(
    _stage_a_body,
    grid=(_GRID,),
    in_specs=[_spec(_B4, 128)] * 3 + [_const((2, 128, 128))],
    out_specs=[_spec(_B4, 128), _spec(_B4, 256), _spec(_B4, 128)],
    out_shape=[_f32((_R4, 128)), _f32((_R4, 256)), _f32((_R4, 128))],
)

_stage_b = pl.pallas_call(
    _stage_b_body,
    grid=(_GRID,),
    in_specs=[_spec(_B4, 128)] * 4 + [_spec(_B4, 256),
              _const((8, 128, 128)), _const((4, 256))],
    out_specs=[_spec(_B4, 256)] * 4,
    out_shape=[_f32((_R4, 256))] * 4,
)

_stage_c = pl.pallas_call(
    _stage_c_body,
    grid=(_GRID,),
    in_specs=[_spec(_B8, 128)] * 13 + [_const((32, 128, 128)),
              _const((1024, 128)), _const((1024, 128)), _const((1, 128))],
    out_specs=[_spec(_B8, 128)] * 2,
    out_shape=[_f32((_R8, 128))] * 2,
)

_stage_d = pl.pallas_call(
    _stage_d_body,
    grid=(_GRID,),
    in_specs=[_spec(_B8, 128)] * 7 + [_const((128, 256)), _const((128, 256)),
              _const((1, 256))],
    out_specs=_spec(_B8, 256),
    out_shape=_f32((_R8, 256)),
)


def kernel(x, edge_index, W1, b1, W2, b2, W3, b3):
    e32 = edge_index.astype(jnp.int32)
    ones4 = jnp.ones((EB, 4), jnp.float32)
    zeros4 = jnp.zeros((RPT, 4), jnp.float32)
    zeros8 = jnp.zeros((RPT, 8), jnp.float32)
    w1b, b1p, w2b, b2p, wy3a, wy3b, b3p = _pack_weights(W1, b1, W2, b2, W3, b3)
    xp = jnp.pad(x, ((0, NP - N), (0, 0))).reshape(_R4, 128)

    def halves(a, fc):
        f = a.reshape(-1)
        return (f[:NP * fc].reshape(NP * fc // 128, 128),
                f[NP * fc:].reshape(NP * fc // 128, 128))

    degp = _deg_kernel(e32, ones4, zeros4)
    d0, d1 = halves(degp, 4)
    dinv4, dinv8w, y1 = _stage_a(d0, d1, xp, jnp.asarray(_E8))

    z1p, = _prop4(e32, y1.reshape(NP, 4), zeros4)
    z10, z11 = halves(z1p, 4)
    y2 = _stage_b(z10, z11, y1, dinv4, dinv8w, w1b, b1p)

    z2 = [_prop8(e32, t.reshape(NP, 8), zeros8)[0] for t in y2]
    dinv8 = dinv8w.reshape(_R8, 128)
    z2h = [h for t in z2 for h in halves(t, 8)]
    y2v = [t.reshape(_R8, 128) for t in y2]
    y3a, y3b = _stage_c(*z2h, *y2v, dinv8, w2b, wy3a, wy3b, b2p)

    z3a, = _prop8(e32, y3a.reshape(NP, 8), zeros8)
    z3b, = _prop8(e32, y3b.reshape(NP, 8), zeros8)
    za0, za1 = halves(z3a, 8)
    zb0, zb1 = halves(z3b, 8)
    outw = _stage_d(za0, za1, zb0, zb1, y3a, y3b, dinv8,
                    jnp.asarray(_PA), jnp.asarray(_PB), b3p)
    return outw.reshape(NP, 16)[:N]
